# initial kernel scaffold (unmeasured)
import jax
import jax.numpy as jnp
from jax import lax
from jax.experimental import pallas as pl
from jax.experimental.pallas import tpu as pltpu

N_DEV = 4
SQ = 256
SKV = 4096
HQ = 32
HL = 8
DH = 128
QB = 64
NQB = 4
GSZ = QB * NQB
NG = SKV // GSZ
SCALE = 0.08838834764831843

_sem_signal = getattr(pl, "semaphore_signal", None) or pltpu.semaphore_signal
_sem_wait = getattr(pl, "semaphore_wait", None) or pltpu.semaphore_wait
_DevIdType = getattr(pl, "DeviceIdType", None) or pltpu.DeviceIdType
_CompilerParams = getattr(pltpu, "CompilerParams", None) or getattr(
    pltpu, "TPUCompilerParams"
)


def _body(x_ref, wq_ref, k_ref, v_ref, wo_ref, out_ref,
          q_full, comm_q, k_scr, v_scr, o_scr, l_scr, comm_o, comm_l,
          comm_ar, ag_send, ag_recv, k_sems, v_sems,
          rso_send, rso_recv, rsl_send, rsl_recv, ar_send, ar_recv):
    my = lax.axis_index("i")
    right = lax.rem(my + 1, N_DEV)
    left = lax.rem(my + N_DEV - 1, N_DEV)

    def kv_copy(ref, scr, sems, g, slot):
        return pltpu.make_async_copy(
            ref.at[pl.ds(g * GSZ, GSZ)], scr.at[slot], sems.at[slot])

    kv_copy(k_ref, k_scr, k_sems, 0, 0).start()
    kv_copy(v_ref, v_scr, v_sems, 0, 0).start()

    q = jnp.dot(x_ref[...].astype(jnp.bfloat16),
                wq_ref[...].astype(jnp.bfloat16),
                preferred_element_type=jnp.float32)
    q = jnp.swapaxes(q.reshape(SQ, HL, DH), 0, 1).astype(jnp.bfloat16)
    comm_q[0, :, :, :] = q
    q_full[pl.ds(my * HL, HL)] = q

    bsem = pltpu.get_barrier_semaphore()
    for nbr in (left, right):
        _sem_signal(bsem, inc=1, device_id=(nbr,),
                    device_id_type=_DevIdType.MESH)
    _sem_wait(bsem, 2)

    for h in range(N_DEV - 1):
        rdma = pltpu.make_async_remote_copy(
            src_ref=comm_q.at[h], dst_ref=comm_q.at[h + 1],
            send_sem=ag_send.at[h], recv_sem=ag_recv.at[h],
            device_id=(right,), device_id_type=_DevIdType.MESH)
        rdma.start()
        rdma.wait()
        origin = lax.rem(my - h - 1 + 2 * N_DEV, N_DEV)
        q_full[pl.ds(origin * HL, HL)] = comm_q[h + 1]

    o_scr[...] = jnp.zeros(o_scr.shape, o_scr.dtype)
    l_scr[...] = jnp.zeros(l_scr.shape, l_scr.dtype)

    def g_body(g, carry):
        slot = lax.rem(g, 2)
        nxt = lax.rem(g + 1, 2)

        @pl.when(g < NG - 1)
        def _():
            kv_copy(k_ref, k_scr, k_sems, g + 1, nxt).start()
            kv_copy(v_ref, v_scr, v_sems, g + 1, nxt).start()

        kv_copy(k_ref, k_scr, k_sems, g, slot).wait()
        kv_copy(v_ref, v_scr, v_sems, g, slot).wait()
        kb = k_scr[slot].astype(jnp.bfloat16)
        vb = v_scr[slot].astype(jnp.bfloat16)
        for r in range(NQB):
            qr = q_full[:, r * QB:(r + 1) * QB, :]
            kr = kb[r * QB:(r + 1) * QB]
            s = lax.dot_general(qr, kr, (((2,), (2,)), ((0,), (1,))),
                                preferred_element_type=jnp.float32)
            w = jnp.exp(s * SCALE)
            l_scr[:, r * QB:(r + 1) * QB] += jnp.sum(w, axis=-1)
            vr = vb[r * QB:(r + 1) * QB]
            o = lax.dot_general(w.astype(jnp.bfloat16), vr,
                                (((2,), (0,)), ((0,), (1,))),
                                preferred_element_type=jnp.float32)
            o_scr[:, r * QB:(r + 1) * QB, :] += o
        return carry

    lax.fori_loop(0, NG, g_body, 0)

    for t in range(N_DEV - 1):
        h_send = lax.rem(my - 1 - t + 2 * N_DEV, N_DEV)
        if t > 0:
            o_scr[pl.ds(h_send * HL, HL)] += comm_o[t - 1]
            l_scr[pl.ds(h_send * HL, HL)] += comm_l[t - 1]
        ro = pltpu.make_async_remote_copy(
            src_ref=o_scr.at[pl.ds(h_send * HL, HL)], dst_ref=comm_o.at[t],
            send_sem=rso_send.at[t], recv_sem=rso_recv.at[t],
            device_id=(right,), device_id_type=_DevIdType.MESH)
        rl = pltpu.make_async_remote_copy(
            src_ref=l_scr.at[pl.ds(h_send * HL, HL)], dst_ref=comm_l.at[t],
            send_sem=rsl_send.at[t], recv_sem=rsl_recv.at[t],
            device_id=(right,), device_id_type=_DevIdType.MESH)
        ro.start()
        rl.start()
        ro.wait()
        rl.wait()

    o_mine = o_scr[pl.ds(my * HL, HL)] + comm_o[N_DEV - 2]
    l_mine = l_scr[pl.ds(my * HL, HL)] + comm_l[N_DEV - 2]
    ctx = o_mine / l_mine[:, :, None]
    ctx = jnp.swapaxes(ctx, 0, 1).reshape(SQ, HL * DH).astype(jnp.bfloat16)
    part = jnp.dot(ctx, wo_ref[...].astype(jnp.bfloat16),
                   preferred_element_type=jnp.float32)

    comm_ar[0, :, :] = part
    for h in range(N_DEV - 1):
        rdma = pltpu.make_async_remote_copy(
            src_ref=comm_ar.at[h], dst_ref=comm_ar.at[h + 1],
            send_sem=ar_send.at[h], recv_sem=ar_recv.at[h],
            device_id=(right,), device_id_type=_DevIdType.MESH)
        rdma.start()
        rdma.wait()
    out_ref[...] = (comm_ar[0] + comm_ar[1]) + (comm_ar[2] + comm_ar[3])


def kernel(x, Wq, K_ext, V_ext, Wo):
    x2 = x.reshape(SQ, x.shape[-1])
    k3 = K_ext.reshape(SKV, HQ, DH)
    v3 = V_ext.reshape(SKV, HQ, DH)
    out = pl.pallas_call(
        _body,
        out_shape=jax.ShapeDtypeStruct((SQ, HL * DH), jnp.float32),
        in_specs=[
            pl.BlockSpec(memory_space=pltpu.VMEM),
            pl.BlockSpec(memory_space=pltpu.VMEM),
            pl.BlockSpec(memory_space=pltpu.ANY),
            pl.BlockSpec(memory_space=pltpu.ANY),
            pl.BlockSpec(memory_space=pltpu.VMEM),
        ],
        out_specs=pl.BlockSpec(memory_space=pltpu.VMEM),
        scratch_shapes=[
            pltpu.VMEM((HQ, SQ, DH), jnp.bfloat16),
            pltpu.VMEM((N_DEV, HL, SQ, DH), jnp.bfloat16),
            pltpu.VMEM((2, GSZ, HQ, DH), jnp.float32),
            pltpu.VMEM((2, GSZ, HQ, DH), jnp.float32),
            pltpu.VMEM((HQ, SQ, DH), jnp.float32),
            pltpu.VMEM((HQ, SQ), jnp.float32),
            pltpu.VMEM((N_DEV - 1, HL, SQ, DH), jnp.float32),
            pltpu.VMEM((N_DEV - 1, HL, SQ), jnp.float32),
            pltpu.VMEM((N_DEV, SQ, HL * DH), jnp.float32),
            pltpu.SemaphoreType.DMA((N_DEV - 1,)),
            pltpu.SemaphoreType.DMA((N_DEV - 1,)),
            pltpu.SemaphoreType.DMA((2,)),
            pltpu.SemaphoreType.DMA((2,)),
            pltpu.SemaphoreType.DMA((N_DEV - 1,)),
            pltpu.SemaphoreType.DMA((N_DEV - 1,)),
            pltpu.SemaphoreType.DMA((N_DEV - 1,)),
            pltpu.SemaphoreType.DMA((N_DEV - 1,)),
            pltpu.SemaphoreType.DMA((N_DEV - 1,)),
            pltpu.SemaphoreType.DMA((N_DEV - 1,)),
        ],
        compiler_params=_CompilerParams(collective_id=0),
    )(x2, Wq, k3, v3, Wo)
    return out.reshape(1, SQ, HL * DH)


# baseline (device time: 233337 ns/iter reference)
import jax
import jax.numpy as jnp
from jax import lax
from jax.experimental import pallas as pl
from jax.experimental.pallas import tpu as pltpu

N_DEV = 4
SQ = 256
SKV = 4096
HQ = 32
HL = 8
DH = 128
QB = 64
NQB = 4
GSZ = QB * NQB
NG = SKV // GSZ
SCALE = 0.08838834764831843

_sem_signal = getattr(pl, "semaphore_signal", None) or pltpu.semaphore_signal
_sem_wait = getattr(pl, "semaphore_wait", None) or pltpu.semaphore_wait
_DevIdType = getattr(pl, "DeviceIdType", None) or pltpu.DeviceIdType
_CompilerParams = getattr(pltpu, "CompilerParams", None) or getattr(
    pltpu, "TPUCompilerParams"
)


def _body(x_ref, wq_ref, k_ref, v_ref, wo_ref, out_ref,
          q_full, comm_q, k_scr, v_scr, o_scr, l_scr, comm_o, comm_l,
          comm_ar, ag_send, ag_recv, k_sems, v_sems,
          rso_send, rso_recv, rsl_send, rsl_recv, ar_send, ar_recv):
    my = lax.axis_index("i")
    right = lax.rem(my + 1, N_DEV)
    left = lax.rem(my + N_DEV - 1, N_DEV)

    def kv_copy(ref, scr, sems, g, slot):
        return pltpu.make_async_copy(
            ref.at[pl.ds(g * GSZ, GSZ)], scr.at[slot], sems.at[slot])

    kv_copy(k_ref, k_scr, k_sems, 0, 0).start()
    kv_copy(v_ref, v_scr, v_sems, 0, 0).start()

    q = jnp.dot(x_ref[...].astype(jnp.bfloat16),
                wq_ref[...].astype(jnp.bfloat16),
                preferred_element_type=jnp.float32)
    q = jnp.swapaxes(q.reshape(SQ, HL, DH), 0, 1).astype(jnp.bfloat16)
    comm_q[0, :, :, :] = q
    q_full[pl.ds(my * HL, HL)] = q

    bsem = pltpu.get_barrier_semaphore()
    for nbr in (left, right):
        _sem_signal(bsem, inc=1, device_id=(nbr,),
                    device_id_type=_DevIdType.MESH)
    _sem_wait(bsem, 2)

    for h in range(N_DEV - 1):
        rdma = pltpu.make_async_remote_copy(
            src_ref=comm_q.at[h], dst_ref=comm_q.at[h + 1],
            send_sem=ag_send.at[h], recv_sem=ag_recv.at[h],
            device_id=(right,), device_id_type=_DevIdType.MESH)
        rdma.start()
        rdma.wait()
        origin = lax.rem(my - h - 1 + 2 * N_DEV, N_DEV)
        q_full[pl.ds(origin * HL, HL)] = comm_q[h + 1]

    o_scr[...] = jnp.zeros(o_scr.shape, o_scr.dtype)
    l_scr[...] = jnp.zeros(l_scr.shape, l_scr.dtype)

    def g_body(g, carry):
        slot = lax.rem(g, 2)
        nxt = lax.rem(g + 1, 2)

        @pl.when(g < NG - 1)
        def _():
            kv_copy(k_ref, k_scr, k_sems, g + 1, nxt).start()
            kv_copy(v_ref, v_scr, v_sems, g + 1, nxt).start()

        kv_copy(k_ref, k_scr, k_sems, g, slot).wait()
        kv_copy(v_ref, v_scr, v_sems, g, slot).wait()
        kb = k_scr[slot].astype(jnp.bfloat16)
        vb = v_scr[slot].astype(jnp.bfloat16)
        for r in range(NQB):
            qr = q_full[:, r * QB:(r + 1) * QB, :]
            kr = kb[r * QB:(r + 1) * QB]
            s = lax.dot_general(qr, kr, (((2,), (2,)), ((0,), (1,))),
                                preferred_element_type=jnp.float32)
            w = jnp.exp(s * SCALE)
            l_scr[:, r * QB:(r + 1) * QB] += jnp.sum(w, axis=-1)
            vr = vb[r * QB:(r + 1) * QB]
            o = lax.dot_general(w.astype(jnp.bfloat16), vr,
                                (((2,), (0,)), ((0,), (1,))),
                                preferred_element_type=jnp.float32)
            o_scr[:, r * QB:(r + 1) * QB, :] += o
        return carry

    lax.fori_loop(0, NG, g_body, 0)

    for t in range(N_DEV - 1):
        h_send = lax.rem(my - 1 - t + 2 * N_DEV, N_DEV)
        if t > 0:
            o_scr[pl.ds(h_send * HL, HL)] += comm_o[t - 1]
            l_scr[pl.ds(h_send * HL, HL)] += comm_l[t - 1]
        ro = pltpu.make_async_remote_copy(
            src_ref=o_scr.at[pl.ds(h_send * HL, HL)], dst_ref=comm_o.at[t],
            send_sem=rso_send.at[t], recv_sem=rso_recv.at[t],
            device_id=(right,), device_id_type=_DevIdType.MESH)
        rl = pltpu.make_async_remote_copy(
            src_ref=l_scr.at[pl.ds(h_send * HL, HL)], dst_ref=comm_l.at[t],
            send_sem=rsl_send.at[t], recv_sem=rsl_recv.at[t],
            device_id=(right,), device_id_type=_DevIdType.MESH)
        ro.start()
        rl.start()
        ro.wait()
        rl.wait()

    o_mine = o_scr[pl.ds(my * HL, HL)] + comm_o[N_DEV - 2]
    l_mine = l_scr[pl.ds(my * HL, HL)] + comm_l[N_DEV - 2]
    ctx = o_mine / l_mine[:, :, None]
    ctx = jnp.swapaxes(ctx, 0, 1).reshape(SQ, HL * DH).astype(jnp.bfloat16)
    part = jnp.dot(ctx, wo_ref[...].astype(jnp.bfloat16),
                   preferred_element_type=jnp.float32)

    comm_ar[0, :, :] = part
    for h in range(N_DEV - 1):
        rdma = pltpu.make_async_remote_copy(
            src_ref=comm_ar.at[h], dst_ref=comm_ar.at[h + 1],
            send_sem=ar_send.at[h], recv_sem=ar_recv.at[h],
            device_id=(right,), device_id_type=_DevIdType.MESH)
        rdma.start()
        rdma.wait()
    out_ref[...] = (comm_ar[0] + comm_ar[1]) + (comm_ar[2] + comm_ar[3])


def kernel(x, Wq, K_ext, V_ext, Wo):
    x2 = x.reshape(SQ, x.shape[-1])
    k3 = K_ext.reshape(SKV, HQ, DH)
    v3 = V_ext.reshape(SKV, HQ, DH)
    out = pl.pallas_call(
        _body,
        out_shape=jax.ShapeDtypeStruct((SQ, HL * DH), jnp.float32),
        in_specs=[
            pl.BlockSpec(memory_space=pltpu.VMEM),
            pl.BlockSpec(memory_space=pltpu.VMEM),
            pl.BlockSpec(memory_space=pl.ANY),
            pl.BlockSpec(memory_space=pl.ANY),
            pl.BlockSpec(memory_space=pltpu.VMEM),
        ],
        out_specs=pl.BlockSpec(memory_space=pltpu.VMEM),
        scratch_shapes=[
            pltpu.VMEM((HQ, SQ, DH), jnp.bfloat16),
            pltpu.VMEM((N_DEV, HL, SQ, DH), jnp.bfloat16),
            pltpu.VMEM((2, GSZ, HQ, DH), jnp.float32),
            pltpu.VMEM((2, GSZ, HQ, DH), jnp.float32),
            pltpu.VMEM((HQ, SQ, DH), jnp.float32),
            pltpu.VMEM((HQ, SQ), jnp.float32),
            pltpu.VMEM((N_DEV - 1, HL, SQ, DH), jnp.float32),
            pltpu.VMEM((N_DEV - 1, HL, SQ), jnp.float32),
            pltpu.VMEM((N_DEV, SQ, HL * DH), jnp.float32),
            pltpu.SemaphoreType.DMA((N_DEV - 1,)),
            pltpu.SemaphoreType.DMA((N_DEV - 1,)),
            pltpu.SemaphoreType.DMA((2,)),
            pltpu.SemaphoreType.DMA((2,)),
            pltpu.SemaphoreType.DMA((N_DEV - 1,)),
            pltpu.SemaphoreType.DMA((N_DEV - 1,)),
            pltpu.SemaphoreType.DMA((N_DEV - 1,)),
            pltpu.SemaphoreType.DMA((N_DEV - 1,)),
            pltpu.SemaphoreType.DMA((N_DEV - 1,)),
            pltpu.SemaphoreType.DMA((N_DEV - 1,)),
        ],
        compiler_params=_CompilerParams(
            collective_id=0, vmem_limit_bytes=100 * 1024 * 1024),
    )(x2, Wq, k3, v3, Wo)
    return out.reshape(1, SQ, HL * DH)


# device time: 198000 ns/iter; 1.1785x vs baseline; 1.1785x over previous
import jax
import jax.numpy as jnp
from jax import lax
from jax.experimental import pallas as pl
from jax.experimental.pallas import tpu as pltpu

N_DEV = 4
SQ = 256
SKV = 4096
HQ = 32
HL = 8
DH = 128
QB = 64
NQB = 4
GSZ = QB * NQB
NG = SKV // GSZ
SCALE = 0.08838834764831843

_sem_signal = getattr(pl, "semaphore_signal", None) or pltpu.semaphore_signal
_sem_wait = getattr(pl, "semaphore_wait", None) or pltpu.semaphore_wait
_DevIdType = getattr(pl, "DeviceIdType", None) or pltpu.DeviceIdType
_CompilerParams = getattr(pltpu, "CompilerParams", None) or getattr(
    pltpu, "TPUCompilerParams"
)


def _body(x_ref, wq_ref, k_ref, v_ref, wo_ref, out_ref,
          q_full, comm_q, k_scr, v_scr, o_scr, l_scr, comm_o, comm_l,
          so_stage, sl_stage, comm_ar, ag_send, ag_recv, k_sems, v_sems,
          rso_send, rso_recv, rsl_send, rsl_recv, ar_send, ar_recv):
    my = lax.axis_index("i")
    right = lax.rem(my + 1, N_DEV)
    left = lax.rem(my + N_DEV - 1, N_DEV)

    def kv_copy(ref, scr, sems, g, slot):
        return pltpu.make_async_copy(
            ref.at[pl.ds(g * GSZ, GSZ)], scr.at[slot], sems.at[slot])

    kv_copy(k_ref, k_scr, k_sems, 0, 0).start()
    kv_copy(v_ref, v_scr, v_sems, 0, 0).start()

    q = jnp.dot(x_ref[...].astype(jnp.bfloat16),
                wq_ref[...].astype(jnp.bfloat16),
                preferred_element_type=jnp.float32)
    q = jnp.swapaxes(q.reshape(SQ, HL, DH), 0, 1).astype(jnp.bfloat16)
    comm_q[0, :, :, :] = q
    q_full[pl.ds(my * HL, HL)] = q

    bsem = pltpu.get_barrier_semaphore()
    for nbr in (left, right):
        _sem_signal(bsem, inc=1, device_id=(nbr,),
                    device_id_type=_DevIdType.MESH)
    _sem_wait(bsem, 2)

    for h in range(N_DEV - 1):
        rdma = pltpu.make_async_remote_copy(
            src_ref=comm_q.at[h], dst_ref=comm_q.at[h + 1],
            send_sem=ag_send.at[h], recv_sem=ag_recv.at[h],
            device_id=(right,), device_id_type=_DevIdType.MESH)
        rdma.start()
        rdma.wait()
        origin = lax.rem(my - h - 1 + 2 * N_DEV, N_DEV)
        q_full[pl.ds(origin * HL, HL)] = comm_q[h + 1]

    o_scr[...] = jnp.zeros(o_scr.shape, o_scr.dtype)
    l_scr[...] = jnp.zeros(l_scr.shape, l_scr.dtype)

    def g_body(g, carry):
        slot = lax.rem(g, 2)
        nxt = lax.rem(g + 1, 2)

        @pl.when(g < NG - 1)
        def _():
            kv_copy(k_ref, k_scr, k_sems, g + 1, nxt).start()
            kv_copy(v_ref, v_scr, v_sems, g + 1, nxt).start()

        kv_copy(k_ref, k_scr, k_sems, g, slot).wait()
        kv_copy(v_ref, v_scr, v_sems, g, slot).wait()
        kb = k_scr[slot].astype(jnp.bfloat16)
        vb = v_scr[slot].astype(jnp.bfloat16)
        for r in range(NQB):
            qr = q_full[:, r * QB:(r + 1) * QB, :]
            kr = kb[r * QB:(r + 1) * QB]
            s = lax.dot_general(qr, kr, (((2,), (2,)), ((0,), (1,))),
                                preferred_element_type=jnp.float32)
            w = jnp.exp(s * SCALE)
            l_scr[:, r * QB:(r + 1) * QB] += jnp.sum(w, axis=-1)
            vr = vb[r * QB:(r + 1) * QB]
            o = lax.dot_general(w.astype(jnp.bfloat16), vr,
                                (((2,), (0,)), ((0,), (1,))),
                                preferred_element_type=jnp.float32)
            o_scr[:, r * QB:(r + 1) * QB, :] += o
        return carry

    lax.fori_loop(0, NG, g_body, 0)

    for t in range(N_DEV - 1):
        h_send = lax.rem(my - 1 - t + 2 * N_DEV, N_DEV)
        ov = o_scr[pl.ds(h_send * HL, HL)]
        lv = l_scr[pl.ds(h_send * HL, HL)]
        if t > 0:
            ov = ov + comm_o[t - 1].astype(jnp.float32)
            lv = lv + comm_l[t - 1].astype(jnp.float32)
        so_stage[t] = ov.astype(jnp.bfloat16)
        sl_stage[t] = lv.astype(jnp.bfloat16)
        ro = pltpu.make_async_remote_copy(
            src_ref=so_stage.at[t], dst_ref=comm_o.at[t],
            send_sem=rso_send.at[t], recv_sem=rso_recv.at[t],
            device_id=(right,), device_id_type=_DevIdType.MESH)
        rl = pltpu.make_async_remote_copy(
            src_ref=sl_stage.at[t], dst_ref=comm_l.at[t],
            send_sem=rsl_send.at[t], recv_sem=rsl_recv.at[t],
            device_id=(right,), device_id_type=_DevIdType.MESH)
        ro.start()
        rl.start()
        ro.wait()
        rl.wait()

    o_mine = (o_scr[pl.ds(my * HL, HL)]
              + comm_o[N_DEV - 2].astype(jnp.float32))
    l_mine = (l_scr[pl.ds(my * HL, HL)]
              + comm_l[N_DEV - 2].astype(jnp.float32))
    ctx = o_mine / l_mine[:, :, None]
    ctx = jnp.swapaxes(ctx, 0, 1).reshape(SQ, HL * DH).astype(jnp.bfloat16)
    part = jnp.dot(ctx, wo_ref[...].astype(jnp.bfloat16),
                   preferred_element_type=jnp.float32)

    comm_ar[0, :, :] = part.astype(jnp.bfloat16)
    for h in range(N_DEV - 1):
        rdma = pltpu.make_async_remote_copy(
            src_ref=comm_ar.at[h], dst_ref=comm_ar.at[h + 1],
            send_sem=ar_send.at[h], recv_sem=ar_recv.at[h],
            device_id=(right,), device_id_type=_DevIdType.MESH)
        rdma.start()
        rdma.wait()
    out_ref[...] = part + (comm_ar[1].astype(jnp.float32)
                           + (comm_ar[2].astype(jnp.float32)
                              + comm_ar[3].astype(jnp.float32)))


def kernel(x, Wq, K_ext, V_ext, Wo):
    x2 = x.reshape(SQ, x.shape[-1])
    k3 = K_ext.reshape(SKV, HQ, DH)
    v3 = V_ext.reshape(SKV, HQ, DH)
    out = pl.pallas_call(
        _body,
        out_shape=jax.ShapeDtypeStruct((SQ, HL * DH), jnp.float32),
        in_specs=[
            pl.BlockSpec(memory_space=pltpu.VMEM),
            pl.BlockSpec(memory_space=pltpu.VMEM),
            pl.BlockSpec(memory_space=pl.ANY),
            pl.BlockSpec(memory_space=pl.ANY),
            pl.BlockSpec(memory_space=pltpu.VMEM),
        ],
        out_specs=pl.BlockSpec(memory_space=pltpu.VMEM),
        scratch_shapes=[
            pltpu.VMEM((HQ, SQ, DH), jnp.bfloat16),
            pltpu.VMEM((N_DEV, HL, SQ, DH), jnp.bfloat16),
            pltpu.VMEM((2, GSZ, HQ, DH), jnp.float32),
            pltpu.VMEM((2, GSZ, HQ, DH), jnp.float32),
            pltpu.VMEM((HQ, SQ, DH), jnp.float32),
            pltpu.VMEM((HQ, SQ), jnp.float32),
            pltpu.VMEM((N_DEV - 1, HL, SQ, DH), jnp.bfloat16),
            pltpu.VMEM((N_DEV - 1, HL, SQ), jnp.bfloat16),
            pltpu.VMEM((N_DEV - 1, HL, SQ, DH), jnp.bfloat16),
            pltpu.VMEM((N_DEV - 1, HL, SQ), jnp.bfloat16),
            pltpu.VMEM((N_DEV, SQ, HL * DH), jnp.bfloat16),
            pltpu.SemaphoreType.DMA((N_DEV - 1,)),
            pltpu.SemaphoreType.DMA((N_DEV - 1,)),
            pltpu.SemaphoreType.DMA((2,)),
            pltpu.SemaphoreType.DMA((2,)),
            pltpu.SemaphoreType.DMA((N_DEV - 1,)),
            pltpu.SemaphoreType.DMA((N_DEV - 1,)),
            pltpu.SemaphoreType.DMA((N_DEV - 1,)),
            pltpu.SemaphoreType.DMA((N_DEV - 1,)),
            pltpu.SemaphoreType.DMA((N_DEV - 1,)),
            pltpu.SemaphoreType.DMA((N_DEV - 1,)),
        ],
        compiler_params=_CompilerParams(
            collective_id=0, vmem_limit_bytes=100 * 1024 * 1024),
    )(x2, Wq, k3, v3, Wo)
    return out.reshape(1, SQ, HL * DH)


# device time: 96513 ns/iter; 2.4177x vs baseline; 2.0515x over previous
import os

import jax
import jax.numpy as jnp
from jax import lax
from jax.experimental import pallas as pl
from jax.experimental.pallas import tpu as pltpu

N_DEV = 4
SQ = 256
QS = SQ // N_DEV
SKV = 4096
HQ = 32
HL = 8
DH = 128
QB = 64
NQB = 4
GSZ = QB * NQB
NG = SKV // GSZ
NSLOT = 4
SCALE = 0.08838834764831843
_ABLATE = os.environ.get("KERNEL_ABLATE", "")

_sem_signal = getattr(pl, "semaphore_signal", None) or pltpu.semaphore_signal
_sem_wait = getattr(pl, "semaphore_wait", None) or pltpu.semaphore_wait
_DevIdType = getattr(pl, "DeviceIdType", None) or pltpu.DeviceIdType
_CompilerParams = getattr(pltpu, "CompilerParams", None) or getattr(
    pltpu, "TPUCompilerParams"
)
_COMM = "nocomm" not in _ABLATE


def _body(x_ref, wq_ref, k_ref, v_ref, wo_ref, out_ref,
          q_full, k_scr, v_scr, o_scr, l_scr, comm_o, comm_l,
          so_stage, sl_stage, part_scr, ar_rsr, ar_rss, ar_agr, ar_ags,
          ag_send, ag_recv, k_sems, v_sems, rso_send, rso_recv,
          rsl_send, rsl_recv, ars_send, ars_recv, ara_send, ara_recv):
    my = lax.axis_index("i")
    peers = [lax.rem(my + d, N_DEV) for d in range(1, N_DEV)]

    def kv_copy(ref, scr, sems, g, slot):
        return pltpu.make_async_copy(
            ref.at[pl.ds(g * GSZ, GSZ)], scr.at[slot], sems.at[slot])

    for g0 in range(NSLOT):
        kv_copy(k_ref, k_scr, k_sems, g0, g0).start()
        kv_copy(v_ref, v_scr, v_sems, g0, g0).start()

    q = jnp.dot(x_ref[...].astype(jnp.bfloat16),
                wq_ref[...].astype(jnp.bfloat16),
                preferred_element_type=jnp.float32)
    q = jnp.swapaxes(q.reshape(SQ, HL, DH), 0, 1).astype(jnp.bfloat16)
    q_full[pl.ds(my * HL, HL)] = q

    bsem = pltpu.get_barrier_semaphore()
    for p in peers:
        _sem_signal(bsem, inc=1, device_id=(p,),
                    device_id_type=_DevIdType.MESH)
    _sem_wait(bsem, N_DEV - 1)

    if _COMM:
        ag_descs = []
        for d in range(1, N_DEV):
            desc = pltpu.make_async_remote_copy(
                src_ref=q_full.at[pl.ds(my * HL, HL)],
                dst_ref=q_full.at[pl.ds(my * HL, HL)],
                send_sem=ag_send.at[d - 1], recv_sem=ag_recv.at[d - 1],
                device_id=(peers[d - 1],), device_id_type=_DevIdType.MESH)
            desc.start()
            ag_descs.append(desc)
        for desc in ag_descs:
            desc.wait()

    o_scr[...] = jnp.zeros(o_scr.shape, o_scr.dtype)
    l_scr[...] = jnp.zeros(l_scr.shape, l_scr.dtype)

    def g_body(g, carry):
        slot = lax.rem(g, NSLOT)
        kv_copy(k_ref, k_scr, k_sems, g, slot).wait()
        kv_copy(v_ref, v_scr, v_sems, g, slot).wait()
        kb = jnp.swapaxes(k_scr[slot].astype(jnp.bfloat16), 0, 1)
        vb = jnp.swapaxes(v_scr[slot].astype(jnp.bfloat16), 0, 1)
        for r in range(NQB if "nomath" not in _ABLATE else 0):
            qr = q_full[:, r * QB:(r + 1) * QB, :]
            kr = kb[:, r * QB:(r + 1) * QB, :]
            s = lax.dot_general(qr, kr, (((2,), (2,)), ((0,), (0,))),
                                preferred_element_type=jnp.float32)
            w = jnp.exp(s * SCALE)
            l_scr[:, r * QB:(r + 1) * QB] += jnp.sum(w, axis=-1)
            vr = vb[:, r * QB:(r + 1) * QB, :]
            o = lax.dot_general(w.astype(jnp.bfloat16), vr,
                                (((2,), (1,)), ((0,), (0,))),
                                preferred_element_type=jnp.float32)
            o_scr[:, r * QB:(r + 1) * QB, :] += o

        @pl.when(g + NSLOT < NG)
        def _():
            kv_copy(k_ref, k_scr, k_sems, g + NSLOT, slot).start()
            kv_copy(v_ref, v_scr, v_sems, g + NSLOT, slot).start()
        return carry

    lax.fori_loop(0, NG, g_body, 0)

    if _COMM:
        rs_descs = []
        for d in range(1, N_DEV):
            hc = peers[d - 1]
            so_stage[d - 1] = o_scr[pl.ds(hc * HL, HL)].astype(jnp.bfloat16)
            sl_stage[d - 1] = l_scr[pl.ds(hc * HL, HL)].astype(jnp.bfloat16)
            for st, cm, ss, rs in (
                    (so_stage, comm_o, rso_send, rso_recv),
                    (sl_stage, comm_l, rsl_send, rsl_recv)):
                desc = pltpu.make_async_remote_copy(
                    src_ref=st.at[d - 1], dst_ref=cm.at[d - 1],
                    send_sem=ss.at[d - 1], recv_sem=rs.at[d - 1],
                    device_id=(peers[d - 1],),
                    device_id_type=_DevIdType.MESH)
                desc.start()
                rs_descs.append(desc)
        for desc in rs_descs:
            desc.wait()
        o_mine = (o_scr[pl.ds(my * HL, HL)]
                  + ((comm_o[0].astype(jnp.float32)
                      + comm_o[1].astype(jnp.float32))
                     + comm_o[2].astype(jnp.float32)))
        l_mine = (l_scr[pl.ds(my * HL, HL)]
                  + ((comm_l[0].astype(jnp.float32)
                      + comm_l[1].astype(jnp.float32))
                     + comm_l[2].astype(jnp.float32)))
    else:
        o_mine = o_scr[pl.ds(my * HL, HL)]
        l_mine = l_scr[pl.ds(my * HL, HL)] + 1.0

    ctx = o_mine / l_mine[:, :, None]
    ctx = jnp.swapaxes(ctx, 0, 1).reshape(SQ, HL * DH).astype(jnp.bfloat16)
    part_scr[...] = jnp.dot(ctx, wo_ref[...].astype(jnp.bfloat16),
                            preferred_element_type=jnp.float32)

    if _COMM:
        ars_descs = []
        for d in range(1, N_DEV):
            qi = peers[d - 1]
            ar_rss[d - 1] = part_scr[pl.ds(qi * QS, QS)].astype(jnp.bfloat16)
            desc = pltpu.make_async_remote_copy(
                src_ref=ar_rss.at[d - 1], dst_ref=ar_rsr.at[d - 1],
                send_sem=ars_send.at[d - 1], recv_sem=ars_recv.at[d - 1],
                device_id=(peers[d - 1],), device_id_type=_DevIdType.MESH)
            desc.start()
            ars_descs.append(desc)
        for desc in ars_descs:
            desc.wait()
        myq = (part_scr[pl.ds(my * QS, QS)]
               + ((ar_rsr[0].astype(jnp.float32)
                   + ar_rsr[1].astype(jnp.float32))
                  + ar_rsr[2].astype(jnp.float32)))
        out_ref[pl.ds(my * QS, QS)] = myq
        ar_ags[0] = myq.astype(jnp.bfloat16)
        ara_descs = []
        for d in range(1, N_DEV):
            desc = pltpu.make_async_remote_copy(
                src_ref=ar_ags.at[0], dst_ref=ar_agr.at[d - 1],
                send_sem=ara_send.at[d - 1], recv_sem=ara_recv.at[d - 1],
                device_id=(peers[d - 1],), device_id_type=_DevIdType.MESH)
            desc.start()
            ara_descs.append(desc)
        for desc in ara_descs:
            desc.wait()
        for d in range(1, N_DEV):
            src_pos = lax.rem(my - d + N_DEV, N_DEV)
            out_ref[pl.ds(src_pos * QS, QS)] = (
                ar_agr[d - 1].astype(jnp.float32))
    else:
        out_ref[...] = part_scr[...]


def kernel(x, Wq, K_ext, V_ext, Wo):
    x2 = x.reshape(SQ, x.shape[-1])
    k3 = K_ext.reshape(SKV, HQ, DH)
    v3 = V_ext.reshape(SKV, HQ, DH)
    out = pl.pallas_call(
        _body,
        out_shape=jax.ShapeDtypeStruct((SQ, HL * DH), jnp.float32),
        in_specs=[
            pl.BlockSpec(memory_space=pltpu.VMEM),
            pl.BlockSpec(memory_space=pltpu.VMEM),
            pl.BlockSpec(memory_space=pl.ANY),
            pl.BlockSpec(memory_space=pl.ANY),
            pl.BlockSpec(memory_space=pltpu.VMEM),
        ],
        out_specs=pl.BlockSpec(memory_space=pltpu.VMEM),
        scratch_shapes=[
            pltpu.VMEM((HQ, SQ, DH), jnp.bfloat16),
            pltpu.VMEM((NSLOT, GSZ, HQ, DH), jnp.float32),
            pltpu.VMEM((NSLOT, GSZ, HQ, DH), jnp.float32),
            pltpu.VMEM((HQ, SQ, DH), jnp.float32),
            pltpu.VMEM((HQ, SQ), jnp.float32),
            pltpu.VMEM((N_DEV - 1, HL, SQ, DH), jnp.bfloat16),
            pltpu.VMEM((N_DEV - 1, HL, SQ), jnp.bfloat16),
            pltpu.VMEM((N_DEV - 1, HL, SQ, DH), jnp.bfloat16),
            pltpu.VMEM((N_DEV - 1, HL, SQ), jnp.bfloat16),
            pltpu.VMEM((SQ, HL * DH), jnp.float32),
            pltpu.VMEM((N_DEV - 1, QS, HL * DH), jnp.bfloat16),
            pltpu.VMEM((N_DEV - 1, QS, HL * DH), jnp.bfloat16),
            pltpu.VMEM((N_DEV - 1, QS, HL * DH), jnp.bfloat16),
            pltpu.VMEM((1, QS, HL * DH), jnp.bfloat16),
            pltpu.SemaphoreType.DMA((N_DEV - 1,)),
            pltpu.SemaphoreType.DMA((N_DEV - 1,)),
            pltpu.SemaphoreType.DMA((NSLOT,)),
            pltpu.SemaphoreType.DMA((NSLOT,)),
            pltpu.SemaphoreType.DMA((N_DEV - 1,)),
            pltpu.SemaphoreType.DMA((N_DEV - 1,)),
            pltpu.SemaphoreType.DMA((N_DEV - 1,)),
            pltpu.SemaphoreType.DMA((N_DEV - 1,)),
            pltpu.SemaphoreType.DMA((N_DEV - 1,)),
            pltpu.SemaphoreType.DMA((N_DEV - 1,)),
            pltpu.SemaphoreType.DMA((N_DEV - 1,)),
            pltpu.SemaphoreType.DMA((N_DEV - 1,)),
        ],
        compiler_params=_CompilerParams(
            collective_id=0, vmem_limit_bytes=120 * 1024 * 1024),
    )(x2, Wq, k3, v3, Wo)
    return out.reshape(1, SQ, HL * DH)
